# Initial kernel scaffold; baseline (speedup 1.0000x reference)
import jax
import jax.numpy as jnp
from jax.experimental import pallas as pl


def kernel(x, top_k, W1, b1, W2, b2):
    raise NotImplementedError("write your pallas kernel here")


# speedup vs baseline: 4.0782x; 4.0782x over previous
import functools

import jax
import jax.numpy as jnp
from jax.experimental import pallas as pl
from jax.experimental.pallas import tpu as pltpu

_BLK = 512


def _router_body(x_ref, w1_ref, b1_ref, w2_ref, b2_ref, w_ref, idx_ref):
    x = x_ref[...]
    h = jnp.maximum(
        jnp.dot(x, w1_ref[...], preferred_element_type=jnp.float32) + b1_ref[...],
        0.0,
    )
    logits = jnp.dot(h, w2_ref[...], preferred_element_type=jnp.float32) + b2_ref[...]
    E = logits.shape[1]
    lane = jax.lax.broadcasted_iota(jnp.int32, logits.shape, 1)
    m1 = jnp.max(logits, axis=1, keepdims=True)
    idx1 = jnp.min(jnp.where(logits == m1, lane, E), axis=1, keepdims=True)
    rest = jnp.where(lane == idx1, -jnp.inf, logits)
    m2 = jnp.max(rest, axis=1, keepdims=True)
    idx2 = jnp.min(jnp.where(rest == m2, lane, E), axis=1, keepdims=True)
    e2 = jnp.exp(m2 - m1)
    p2 = e2 / (1.0 + e2)
    p1 = 1.0 - p2
    w_ref[...] = jnp.where(
        lane == idx1, p1, jnp.where(lane == idx2, p2, 0.0)
    )
    idx_ref[...] = jnp.concatenate([idx1, idx2], axis=1)


def kernel(x, top_k, W1, b1, W2, b2):
    del top_k
    B, D = x.shape
    H = W1.shape[1]
    E = W2.shape[1]
    grid = (B // _BLK,)
    weights, idx = pl.pallas_call(
        _router_body,
        grid=grid,
        in_specs=[
            pl.BlockSpec((_BLK, D), lambda i: (i, 0)),
            pl.BlockSpec((D, H), lambda i: (0, 0)),
            pl.BlockSpec((1, H), lambda i: (0, 0)),
            pl.BlockSpec((H, E), lambda i: (0, 0)),
            pl.BlockSpec((1, E), lambda i: (0, 0)),
        ],
        out_specs=[
            pl.BlockSpec((_BLK, E), lambda i: (i, 0)),
            pl.BlockSpec((_BLK, 2), lambda i: (i, 0)),
        ],
        out_shape=[
            jax.ShapeDtypeStruct((B, E), jnp.float32),
            jax.ShapeDtypeStruct((B, 2), jnp.int32),
        ],
        compiler_params=pltpu.CompilerParams(
            dimension_semantics=("parallel",),
        ),
    )(x, W1, b1.reshape(1, H), W2, b2.reshape(1, E))
    return weights, idx


# speedup vs baseline: 5.2570x; 1.2891x over previous
import functools

import jax
import jax.numpy as jnp
from jax.experimental import pallas as pl
from jax.experimental.pallas import tpu as pltpu

_BLK = 1024


def _router_body(x_ref, w1_ref, b1_ref, w2_ref, b2_ref, w_ref, idx_ref):
    x = x_ref[...]
    h = jnp.maximum(
        jnp.dot(x, w1_ref[...], preferred_element_type=jnp.float32) + b1_ref[...],
        0.0,
    )
    logits = jnp.dot(h, w2_ref[...], preferred_element_type=jnp.float32) + b2_ref[...]
    E = logits.shape[1]
    lane = jax.lax.broadcasted_iota(jnp.int32, logits.shape, 1)
    m1 = jnp.max(logits, axis=1, keepdims=True)
    idx1 = jnp.min(jnp.where(logits == m1, lane, E), axis=1, keepdims=True)
    rest = jnp.where(lane == idx1, -jnp.inf, logits)
    m2 = jnp.max(rest, axis=1, keepdims=True)
    idx2 = jnp.min(jnp.where(rest == m2, lane, E), axis=1, keepdims=True)
    e2 = jnp.exp(m2 - m1)
    p2 = e2 / (1.0 + e2)
    p1 = 1.0 - p2
    w_ref[...] = jnp.where(
        lane == idx1, p1, jnp.where(lane == idx2, p2, 0.0)
    )
    idx_ref[...] = jnp.concatenate([idx1, idx2], axis=1)


def kernel(x, top_k, W1, b1, W2, b2):
    del top_k
    B, D = x.shape
    H = W1.shape[1]
    E = W2.shape[1]
    grid = (B // _BLK,)
    weights, idx = pl.pallas_call(
        _router_body,
        grid=grid,
        in_specs=[
            pl.BlockSpec((_BLK, D), lambda i: (i, 0)),
            pl.BlockSpec((D, H), lambda i: (0, 0)),
            pl.BlockSpec((1, H), lambda i: (0, 0)),
            pl.BlockSpec((H, E), lambda i: (0, 0)),
            pl.BlockSpec((1, E), lambda i: (0, 0)),
        ],
        out_specs=[
            pl.BlockSpec((_BLK, E), lambda i: (i, 0)),
            pl.BlockSpec((_BLK, 2), lambda i: (i, 0)),
        ],
        out_shape=[
            jax.ShapeDtypeStruct((B, E), jnp.float32),
            jax.ShapeDtypeStruct((B, 2), jnp.int32),
        ],
        compiler_params=pltpu.CompilerParams(
            dimension_semantics=("parallel",),
        ),
    )(x, W1, b1.reshape(1, H), W2, b2.reshape(1, E))
    return weights, idx


# speedup vs baseline: 5.5695x; 1.0594x over previous
import functools

import jax
import jax.numpy as jnp
from jax.experimental import pallas as pl
from jax.experimental.pallas import tpu as pltpu

_BLK = 1024


def _router_body(x_ref, w1_ref, b1_ref, w2_ref, b2_ref, w_ref, idx_ref):
    x = x_ref[...]
    h = jnp.maximum(
        jnp.dot(x, w1_ref[...], preferred_element_type=jnp.float32) + b1_ref[...],
        0.0,
    )
    logits = jnp.dot(h, w2_ref[...], preferred_element_type=jnp.float32) + b2_ref[...]
    E = logits.shape[1]
    lane = jax.lax.broadcasted_iota(jnp.int32, logits.shape, 1).astype(jnp.float32)
    m1 = jnp.max(logits, axis=1, keepdims=True)
    idx1 = jnp.min(jnp.where(logits == m1, lane, float(E)), axis=1, keepdims=True)
    rest = jnp.where(lane == idx1, -jnp.inf, logits)
    m2 = jnp.max(rest, axis=1, keepdims=True)
    idx2 = jnp.min(jnp.where(rest == m2, lane, float(E)), axis=1, keepdims=True)
    e2 = jnp.exp(m2 - m1)
    p2 = e2 / (1.0 + e2)
    p1 = 1.0 - p2
    w_ref[...] = jnp.where(
        lane == idx1, p1, jnp.where(lane == idx2, p2, 0.0)
    )
    idx_ref[...] = jnp.concatenate([idx1, idx2], axis=1).astype(jnp.int32)


def kernel(x, top_k, W1, b1, W2, b2):
    del top_k
    B, D = x.shape
    H = W1.shape[1]
    E = W2.shape[1]
    grid = (B // _BLK,)
    weights, idx = pl.pallas_call(
        _router_body,
        grid=grid,
        in_specs=[
            pl.BlockSpec((_BLK, D), lambda i: (i, 0)),
            pl.BlockSpec((D, H), lambda i: (0, 0)),
            pl.BlockSpec((1, H), lambda i: (0, 0)),
            pl.BlockSpec((H, E), lambda i: (0, 0)),
            pl.BlockSpec((1, E), lambda i: (0, 0)),
        ],
        out_specs=[
            pl.BlockSpec((_BLK, E), lambda i: (i, 0)),
            pl.BlockSpec((_BLK, 2), lambda i: (i, 0)),
        ],
        out_shape=[
            jax.ShapeDtypeStruct((B, E), jnp.float32),
            jax.ShapeDtypeStruct((B, 2), jnp.int32),
        ],
        compiler_params=pltpu.CompilerParams(
            dimension_semantics=("parallel",),
        ),
    )(x, W1, b1.reshape(1, H), W2, b2.reshape(1, E))
    return weights, idx


# speedup vs baseline: 6.0835x; 1.0923x over previous
import jax
import jax.numpy as jnp
from jax import lax
from jax.experimental import pallas as pl
from jax.experimental.pallas import tpu as pltpu

_BLK = 1024


def _router_body(x_ref, w1_ref, b1_ref, w2_ref, b2_ref, eye_ref, wt_ref, idxt_ref):
    x = x_ref[...]
    h = jnp.maximum(
        jnp.dot(x, w1_ref[...], preferred_element_type=jnp.float32) + b1_ref[...],
        0.0,
    )
    logits = jnp.dot(h, w2_ref[...], preferred_element_type=jnp.float32) + b2_ref[...]
    E = logits.shape[1]
    lane = lax.broadcasted_iota(jnp.int32, logits.shape, 1).astype(jnp.float32)
    m1 = jnp.max(logits, axis=1, keepdims=True)
    idx1 = jnp.min(jnp.where(logits == m1, lane, float(E)), axis=1, keepdims=True)
    rest = jnp.where(lane == idx1, -jnp.inf, logits)
    m2 = jnp.max(rest, axis=1, keepdims=True)
    idx2 = jnp.min(jnp.where(rest == m2, lane, float(E)), axis=1, keepdims=True)
    e2 = jnp.exp(m2 - m1)
    p2 = e2 / (1.0 + e2)
    p1 = 1.0 - p2
    zero = jnp.zeros_like(p1)
    small = jnp.concatenate(
        [idx1, idx2, p1, p2, zero, zero, zero, zero], axis=1
    )
    small_t = lax.dot_general(
        small,
        eye_ref[...],
        ((( 0,), (0,)), ((), ())),
        preferred_element_type=jnp.float32,
    )
    idxt_ref[...] = small_t.astype(jnp.int32)
    i1r = small_t[0:1, :]
    i2r = small_t[1:2, :]
    p1r = small_t[2:3, :]
    p2r = small_t[3:4, :]
    sub = lax.broadcasted_iota(jnp.int32, (E, small.shape[0]), 0).astype(jnp.float32)
    wt_ref[...] = jnp.where(sub == i1r, p1r, jnp.where(sub == i2r, p2r, 0.0))


def kernel(x, top_k, W1, b1, W2, b2):
    del top_k
    B, D = x.shape
    H = W1.shape[1]
    E = W2.shape[1]
    grid = (B // _BLK,)
    eye = jnp.eye(_BLK, dtype=jnp.float32)
    wt, idxt = pl.pallas_call(
        _router_body,
        grid=grid,
        in_specs=[
            pl.BlockSpec((_BLK, D), lambda i: (i, 0)),
            pl.BlockSpec((D, H), lambda i: (0, 0)),
            pl.BlockSpec((1, H), lambda i: (0, 0)),
            pl.BlockSpec((H, E), lambda i: (0, 0)),
            pl.BlockSpec((1, E), lambda i: (0, 0)),
            pl.BlockSpec((_BLK, _BLK), lambda i: (0, 0)),
        ],
        out_specs=[
            pl.BlockSpec((E, _BLK), lambda i: (0, i)),
            pl.BlockSpec((8, _BLK), lambda i: (0, i)),
        ],
        out_shape=[
            jax.ShapeDtypeStruct((E, B), jnp.float32),
            jax.ShapeDtypeStruct((8, B), jnp.int32),
        ],
        compiler_params=pltpu.CompilerParams(
            dimension_semantics=("parallel",),
        ),
    )(x, W1, b1.reshape(1, H), W2, b2.reshape(1, E), eye)
    return wt.T, idxt[:2, :].T
